# Initial kernel scaffold; baseline (speedup 1.0000x reference)
import jax
import jax.numpy as jnp
from jax.experimental import pallas as pl


def kernel(inp, gate_w, gate_b, W1, W2, ln_g, ln_b, bias):
    raise NotImplementedError("write your pallas kernel here")


# speedup vs baseline: 5.2367x; 5.2367x over previous
import functools

import jax
import jax.numpy as jnp
from jax import lax
from jax.experimental import pallas as pl
from jax.experimental.pallas import tpu as pltpu

_pcall = functools.partial(pl.pallas_call)

_E = 8
_NEG = -3.0e38


def _gate_body(x_ref, gw_ref, gb_ref, c_ref):
    x = x_ref[...]
    gw = gw_ref[...]
    logits = lax.dot_general(x, gw, (((1,), (1,)), ((), ())),
                             preferred_element_type=jnp.float32)
    logits = logits + gb_ref[...]
    iota = lax.broadcasted_iota(jnp.int32, logits.shape, 1)
    m1 = jnp.max(logits, axis=1, keepdims=True)
    i1 = jnp.min(jnp.where(logits == m1, iota, _E), axis=1, keepdims=True)
    l2 = jnp.where(iota == i1, _NEG, logits)
    m2 = jnp.max(l2, axis=1, keepdims=True)
    i2 = jnp.min(jnp.where(l2 == m2, iota, _E), axis=1, keepdims=True)
    p2 = jnp.exp(m2 - m1)
    denom = 1.0 + p2
    g1 = 1.0 / denom
    g2 = p2 / denom
    c_ref[...] = jnp.where(iota == i1, g1, 0.0) + jnp.where(iota == i2, g2, 0.0)


def _gating(flat, gate_w, gate_b):
    n, d = flat.shape
    return _pcall(
        _gate_body,
        out_shape=jax.ShapeDtypeStruct((n, _E), jnp.float32),
    )(flat, gate_w, gate_b.reshape(1, _E))


def _moe_body(x_ref, w1_ref, w2_ref, c_ref, lng_ref, lnb_ref, o_ref, acc_ref):
    e = pl.program_id(1)
    kb = pl.program_id(2)

    @pl.when(jnp.logical_and(e == 0, kb == 0))
    def _init():
        acc_ref[...] = jnp.zeros_like(acc_ref)

    x = x_ref[...]
    h = lax.dot_general(x, w1_ref[0], (((1,), (1,)), ((), ())),
                        preferred_element_type=jnp.float32)
    h = 0.5 * h * (1.0 + lax.erf(h * 0.7071067811865476))
    y = lax.dot_general(h, w2_ref[0], (((1,), (1,)), ((), ())),
                        preferred_element_type=jnp.float32)
    sel = lax.broadcasted_iota(jnp.int32, c_ref.shape, 1) == e
    ce = jnp.sum(jnp.where(sel, c_ref[...], 0.0), axis=1, keepdims=True)
    acc_ref[...] += y * ce

    @pl.when(jnp.logical_and(e == _E - 1, kb == pl.num_programs(2) - 1))
    def _fin():
        a = acc_ref[...] + x
        mu = jnp.mean(a, axis=1, keepdims=True)
        var = jnp.mean((a - mu) ** 2, axis=1, keepdims=True)
        o_ref[...] = (a - mu) * lax.rsqrt(var + 1e-5) * lng_ref[...] + lnb_ref[...]


def _moe(flat, W1, W2, c, ln_g, ln_b, t=1024, kbs=512):
    n, d = flat.shape
    e_, dh, _ = W1.shape
    grid = (n // t, e_, dh // kbs)
    return _pcall(
        _moe_body,
        grid=grid,
        in_specs=[
            pl.BlockSpec((t, d), lambda i, e, k: (i, 0)),
            pl.BlockSpec((1, kbs, d), lambda i, e, k: (e, k, 0)),
            pl.BlockSpec((1, d, kbs), lambda i, e, k: (e, 0, k)),
            pl.BlockSpec((t, _E), lambda i, e, k: (i, 0)),
            pl.BlockSpec((1, d), lambda i, e, k: (0, 0)),
            pl.BlockSpec((1, d), lambda i, e, k: (0, 0)),
        ],
        out_specs=pl.BlockSpec((t, d), lambda i, e, k: (i, 0)),
        out_shape=jax.ShapeDtypeStruct((n, d), jnp.float32),
        scratch_shapes=[pltpu.VMEM((t, d), jnp.float32)],
    )(flat, W1, W2, c, ln_g.reshape(1, d), ln_b.reshape(1, d))


def kernel(inp, gate_w, gate_b, W1, W2, ln_g, ln_b, bias):
    s, b, d = inp.shape
    flat = inp.reshape(s * b, d)
    c = _gating(flat, gate_w, gate_b)
    out = _moe(flat, W1, W2, c, ln_g, ln_b)
    return out.reshape(s, b, d), bias


# speedup vs baseline: 5.4282x; 1.0366x over previous
import functools

import jax
import jax.numpy as jnp
from jax import lax
from jax.experimental import pallas as pl
from jax.experimental.pallas import tpu as pltpu

_pcall = functools.partial(pl.pallas_call)

_E = 8
_NEG = -3.0e38


def _gate_body(x_ref, gw_ref, gb_ref, c_ref):
    x = x_ref[...]
    gw = gw_ref[...]
    logits = lax.dot_general(x, gw, (((1,), (1,)), ((), ())),
                             preferred_element_type=jnp.float32)
    logits = logits + gb_ref[...]
    iota = lax.broadcasted_iota(jnp.int32, logits.shape, 1)
    m1 = jnp.max(logits, axis=1, keepdims=True)
    i1 = jnp.min(jnp.where(logits == m1, iota, _E), axis=1, keepdims=True)
    l2 = jnp.where(iota == i1, _NEG, logits)
    m2 = jnp.max(l2, axis=1, keepdims=True)
    i2 = jnp.min(jnp.where(l2 == m2, iota, _E), axis=1, keepdims=True)
    p2 = jnp.exp(m2 - m1)
    denom = 1.0 + p2
    g1 = 1.0 / denom
    g2 = p2 / denom
    c_ref[...] = jnp.where(iota == i1, g1, 0.0) + jnp.where(iota == i2, g2, 0.0)


def _gating(flat, gate_w, gate_b):
    n, d = flat.shape
    return _pcall(
        _gate_body,
        out_shape=jax.ShapeDtypeStruct((n, _E), jnp.float32),
    )(flat, gate_w, gate_b.reshape(1, _E))


def _moe_body(x_ref, w1_ref, w2_ref, c_ref, lng_ref, lnb_ref, o_ref, acc_ref):
    e = pl.program_id(1)
    kb = pl.program_id(2)

    @pl.when(jnp.logical_and(e == 0, kb == 0))
    def _init():
        acc_ref[...] = jnp.zeros_like(acc_ref)

    x = x_ref[...]
    h = lax.dot_general(x, w1_ref[0], (((1,), (1,)), ((), ())),
                        preferred_element_type=jnp.float32)
    h = 0.5 * h * (1.0 + lax.erf(h * 0.7071067811865476))
    y = lax.dot_general(h, w2_ref[0], (((1,), (1,)), ((), ())),
                        preferred_element_type=jnp.float32)
    sel = lax.broadcasted_iota(jnp.int32, c_ref.shape, 1) == e
    ce = jnp.sum(jnp.where(sel, c_ref[...], 0.0), axis=1, keepdims=True)
    acc_ref[...] += y * ce

    @pl.when(jnp.logical_and(e == _E - 1, kb == pl.num_programs(2) - 1))
    def _fin():
        a = acc_ref[...] + x
        mu = jnp.mean(a, axis=1, keepdims=True)
        var = jnp.mean((a - mu) ** 2, axis=1, keepdims=True)
        o_ref[...] = (a - mu) * lax.rsqrt(var + 1e-5) * lng_ref[...] + lnb_ref[...]


def _moe(flat, W1, W2, c, ln_g, ln_b, t=2048, kbs=512):
    n, d = flat.shape
    e_, dh, _ = W1.shape
    grid = (n // t, e_, dh // kbs)
    return _pcall(
        _moe_body,
        grid=grid,
        in_specs=[
            pl.BlockSpec((t, d), lambda i, e, k: (i, 0)),
            pl.BlockSpec((1, kbs, d), lambda i, e, k: (e, k, 0)),
            pl.BlockSpec((1, d, kbs), lambda i, e, k: (e, 0, k)),
            pl.BlockSpec((t, _E), lambda i, e, k: (i, 0)),
            pl.BlockSpec((1, d), lambda i, e, k: (0, 0)),
            pl.BlockSpec((1, d), lambda i, e, k: (0, 0)),
        ],
        out_specs=pl.BlockSpec((t, d), lambda i, e, k: (i, 0)),
        out_shape=jax.ShapeDtypeStruct((n, d), jnp.float32),
        scratch_shapes=[pltpu.VMEM((t, d), jnp.float32)],
    )(flat, W1, W2, c, ln_g.reshape(1, d), ln_b.reshape(1, d))


def kernel(inp, gate_w, gate_b, W1, W2, ln_g, ln_b, bias):
    s, b, d = inp.shape
    flat = inp.reshape(s * b, d)
    c = _gating(flat, gate_w, gate_b)
    out = _moe(flat, W1, W2, c, ln_g, ln_b)
    return out.reshape(s, b, d), bias
